# Initial kernel scaffold; baseline (speedup 1.0000x reference)
import jax
import jax.numpy as jnp
from jax.experimental import pallas as pl


def kernel(x, W1, W2, W3, W4, W5, L1, L2w, L2b, L3w, L3b, L4w, L4b, L5w, L5b):
    raise NotImplementedError("write your pallas kernel here")


# speedup vs baseline: 7.6584x; 7.6584x over previous
import functools

import jax
import jax.numpy as jnp
from jax import lax
from jax.experimental import pallas as pl
from jax.experimental.pallas import tpu as pltpu
from jax.experimental.pallas import tpu_sc as plsc

B = 8
N = 2048
BN = B * N
KNN = 20
EPS = 1e-5

NC = 2
NS = 16
NW = NC * NS
PPT = BN // NW

_DIMS = (((1,), (1,)), ((), ()))
_DIMS_NT = (((1,), (0,)), ((), ()))


def _lrelu(x):
    return jnp.where(x >= 0, x, 0.2 * x)


_KR = 256
_NBLK = N // _KR


def _sq_norm(xm):
    c = xm.shape[1]
    acc = xm[:, 0:8] * xm[:, 0:8]
    for j in range(1, c // 8):
        g = xm[:, 8 * j:8 * j + 8]
        acc = acc + g * g
    w = acc[:, 0:4] + acc[:, 4:8]
    y = w[:, 0:2] + w[:, 2:4]
    return y[:, 0:1] + y[:, 1:2]


def _knn_body(xr_ref, xf_ref, idx_ref):
    b = pl.program_id(0)
    xf = xf_ref[...]
    xr = xr_ref[...]
    xxc = jnp.transpose(_sq_norm(xf))
    xxr = _sq_norm(xr)
    inner = -2.0 * lax.dot_general(xr, xf, _DIMS, preferred_element_type=jnp.float32)
    d = (-xxc - inner) - xxr
    iota = lax.broadcasted_iota(jnp.int32, (_KR, N), 1)
    lanek = lax.broadcasted_iota(jnp.int32, (_KR, KNN), 1)
    acc = jnp.zeros((_KR, KNN), jnp.int32)
    neg = jnp.float32(-jnp.inf)
    for k in range(KNN):
        m = jnp.max(d, axis=1, keepdims=True)
        am = jnp.min(jnp.where(d >= m, iota, N), axis=1, keepdims=True)
        acc = jnp.where(lanek == k, am + b * N, acc)
        d = jnp.where(iota == am, neg, d)
    idx_ref[...] = jnp.transpose(acc)


@functools.cache
def _knn_call(c):
    return pl.pallas_call(
        _knn_body,
        grid=(B, _NBLK),
        in_specs=[
            pl.BlockSpec((_KR, c), lambda b, i: (b * _NBLK + i, 0)),
            pl.BlockSpec((N, c), lambda b, i: (b, 0)),
        ],
        out_specs=pl.BlockSpec((KNN, _KR), lambda b, i: (0, b * _NBLK + i)),
        out_shape=jax.ShapeDtypeStruct((KNN, BN), jnp.int32),
    )


def _uv_body(x_ref, wa_ref, wb_ref, u_ref, v_ref):
    x = x_ref[...]
    u_ref[...] = lax.dot_general(x, wa_ref[...], _DIMS_NT,
                                 preferred_element_type=jnp.float32)
    v_ref[...] = lax.dot_general(x, wb_ref[...], _DIMS_NT,
                                 preferred_element_type=jnp.float32)


@functools.cache
def _uv_call(c, o):
    return pl.pallas_call(
        _uv_body,
        grid=(B,),
        in_specs=[
            pl.BlockSpec((N, c), lambda b: (b, 0)),
            pl.BlockSpec((c, o), lambda b: (0, 0)),
            pl.BlockSpec((c, o), lambda b: (0, 0)),
        ],
        out_specs=[
            pl.BlockSpec((N, o), lambda b: (b, 0)),
            pl.BlockSpec((N, o), lambda b: (b, 0)),
        ],
        out_shape=[
            jax.ShapeDtypeStruct((BN, o), jnp.float32),
            jax.ShapeDtypeStruct((BN, o), jnp.float32),
        ],
    )


def _sc_cnt(o):
    return 128 if o <= 128 else 64


def _sc_body(o, cnt, u_hbm, v_hbm, idx_hbm, ymax_hbm, part_hbm,
             col_v, v_v, g0, g1, gmax, gsum, gsq, stats, sem0, sem1):
    nj = o // 16
    wid = lax.axis_index("s") * NC + lax.axis_index("c")
    base = wid * PPT
    nsub = PPT // cnt

    def zero_stats(j, _):
        sl = pl.ds(j * 16, 16)
        z = jnp.zeros((16,), jnp.float32)
        stats[0, sl] = z
        stats[1, sl] = z
        return 0

    lax.fori_loop(0, nj, zero_stats, 0)

    def sub(s, _):
        row0 = base + s * cnt
        pltpu.sync_copy(idx_hbm.at[:, pl.ds(row0, cnt)], col_v)
        pltpu.sync_copy(v_hbm.at[pl.ds(row0, cnt), :], v_v)

        bufs = (g0, g1)
        sems = (sem0, sem1)
        d = pltpu.async_copy(u_hbm.at[col_v.at[0]], g0, sem0)
        for k in range(KNN):
            if k + 1 < KNN:
                dn = pltpu.async_copy(u_hbm.at[col_v.at[k + 1]],
                                      bufs[(k + 1) % 2], sems[(k + 1) % 2])
            d.wait()
            buf = bufs[k % 2]
            if k == 0:
                def acc0(r, _):
                    for j in range(nj):
                        sl = pl.ds(j * 16, 16)
                        g = buf[r, sl]
                        gmax[r, sl] = g
                        gsum[r, sl] = g
                        gsq[r, sl] = g * g
                    return 0
                lax.fori_loop(0, cnt, acc0, 0)
            else:
                def acck(r, _, buf=buf):
                    for j in range(nj):
                        sl = pl.ds(j * 16, 16)
                        g = buf[r, sl]
                        gmax[r, sl] = jnp.maximum(gmax[r, sl], g)
                        gsum[r, sl] = gsum[r, sl] + g
                        gsq[r, sl] = gsq[r, sl] + g * g
                    return 0
                lax.fori_loop(0, cnt, acck, 0)
            if k + 1 < KNN:
                d = dn

        kf = jnp.float32(KNN)

        def post(r, _):
            for j in range(nj):
                sl = pl.ds(j * 16, 16)
                vv = v_v[r, sl]
                gs = gsum[r, sl]
                gmax[r, sl] = gmax[r, sl] + vv
                stats[0, sl] = stats[0, sl] + gs + kf * vv
                stats[1, sl] = stats[1, sl] + gsq[r, sl] + 2.0 * vv * gs + kf * vv * vv
            return 0

        lax.fori_loop(0, cnt, post, 0)
        pltpu.sync_copy(gmax, ymax_hbm.at[pl.ds(row0, cnt), :])
        return 0

    lax.fori_loop(0, nsub, sub, 0)
    pltpu.sync_copy(stats, part_hbm.at[wid])


@functools.cache
def _sc_call(o):
    cnt = _sc_cnt(o)
    mesh = plsc.VectorSubcoreMesh(core_axis_name="c", subcore_axis_name="s",
                                  num_cores=NC, num_subcores=NS)
    return pl.kernel(
        functools.partial(_sc_body, o, cnt),
        compiler_params=pltpu.CompilerParams(use_tc_tiling_on_sc=False),
        out_type=[
            jax.ShapeDtypeStruct((BN, o), jnp.float32),
            jax.ShapeDtypeStruct((NW, 2, o), jnp.float32),
        ],
        mesh=mesh,
        scratch_types=[
            pltpu.VMEM((KNN, cnt), jnp.int32),
            pltpu.VMEM((cnt, o), jnp.float32),
            pltpu.VMEM((cnt, o), jnp.float32),
            pltpu.VMEM((cnt, o), jnp.float32),
            pltpu.VMEM((cnt, o), jnp.float32),
            pltpu.VMEM((cnt, o), jnp.float32),
            pltpu.VMEM((cnt, o), jnp.float32),
            pltpu.VMEM((2, o), jnp.float32),
            pltpu.SemaphoreType.DMA,
            pltpu.SemaphoreType.DMA,
        ],
    )


def _sc_gather(u, v, gidx, o):
    return _sc_call(o)(u, v, gidx)


def _fin_body(ymax_ref, part_ref, x_ref):
    part = part_ref[...]
    s1 = jnp.sum(part[:, 0, :], axis=0)
    s2 = jnp.sum(part[:, 1, :], axis=0)
    cnt = jnp.float32(BN * KNN)
    mean = s1 / cnt
    var = s2 / cnt - mean * mean
    y = (ymax_ref[...] - mean[None, :]) / jnp.sqrt(var + EPS)[None, :]
    x_ref[...] = _lrelu(y)


@functools.cache
def _fin_call(o):
    return pl.pallas_call(
        _fin_body,
        grid=(B,),
        in_specs=[
            pl.BlockSpec((N, o), lambda b: (b, 0)),
            pl.BlockSpec((NW, 2, o), lambda b: (0, 0, 0)),
        ],
        out_specs=pl.BlockSpec((N, o), lambda b: (b, 0)),
        out_shape=jax.ShapeDtypeStruct((BN, o), jnp.float32),
    )


def _heada_body(x1_ref, x2_ref, x3_ref, x4_ref, wa_ref, wb_ref, wc_ref, wd_ref,
                z_ref):
    z = lax.dot_general(x1_ref[...], wa_ref[...], _DIMS_NT,
                        preferred_element_type=jnp.float32)
    z = z + lax.dot_general(x2_ref[...], wb_ref[...], _DIMS_NT,
                            preferred_element_type=jnp.float32)
    z = z + lax.dot_general(x3_ref[...], wc_ref[...], _DIMS_NT,
                            preferred_element_type=jnp.float32)
    z = z + lax.dot_general(x4_ref[...], wd_ref[...], _DIMS_NT,
                            preferred_element_type=jnp.float32)
    z_ref[...] = z


@functools.cache
def _heada_call():
    return pl.pallas_call(
        _heada_body,
        grid=(B,),
        in_specs=[
            pl.BlockSpec((N, 64), lambda b: (b, 0)),
            pl.BlockSpec((N, 64), lambda b: (b, 0)),
            pl.BlockSpec((N, 128), lambda b: (b, 0)),
            pl.BlockSpec((N, 256), lambda b: (b, 0)),
            pl.BlockSpec((64, 128), lambda b: (0, 0)),
            pl.BlockSpec((64, 128), lambda b: (0, 0)),
            pl.BlockSpec((128, 128), lambda b: (0, 0)),
            pl.BlockSpec((256, 128), lambda b: (0, 0)),
        ],
        out_specs=pl.BlockSpec((N, 128), lambda b: (b, 0)),
        out_shape=jax.ShapeDtypeStruct((BN, 128), jnp.float32),
    )


def _bn_rows(h):
    m = jnp.mean(h, axis=0, keepdims=True)
    d = h - m
    v = jnp.mean(d * d, axis=0, keepdims=True)
    return d * lax.rsqrt(v + EPS)


def _headb_body(z_ref, l1_ref, l2w_ref, l2b_ref, l3w_ref, l3b_ref,
                l4w_ref, l4b_ref, l5w_ref, l5b_ref, out_ref):
    z = z_ref[...]
    zs = jnp.sum(z, axis=1)
    s1 = jnp.sum(zs, axis=0)
    mean = s1 / jnp.float32(BN)
    dz = z - mean[None, None, :]
    q = jnp.sum(dz * dz, axis=1)
    s2 = jnp.sum(q, axis=0)
    rstd = lax.rsqrt(s2 / jnp.float32(BN) + EPS)
    zmax = jnp.max(z, axis=1)
    p1 = _lrelu((zmax - mean[None, :]) * rstd[None, :])
    h5 = _lrelu(dz * rstd[None, None, :])
    p2 = jnp.sum(h5, axis=1) / jnp.float32(N)
    h = jnp.concatenate([p1, p2], axis=1)
    h = _lrelu(_bn_rows(lax.dot_general(h, l1_ref[...], _DIMS_NT,
                                        preferred_element_type=jnp.float32)))
    h = lax.dot_general(h, l2w_ref[...], _DIMS_NT,
                        preferred_element_type=jnp.float32) + l2b_ref[...]
    h = _lrelu(_bn_rows(h))
    h = lax.dot_general(h, l3w_ref[...], _DIMS_NT,
                        preferred_element_type=jnp.float32) + l3b_ref[...]
    h = lax.dot_general(h, l4w_ref[...], _DIMS_NT,
                        preferred_element_type=jnp.float32) + l4b_ref[...]
    h = lax.dot_general(h, l5w_ref[...], _DIMS_NT,
                        preferred_element_type=jnp.float32) + l5b_ref[...]
    out_ref[...] = h


@functools.cache
def _headb_call():
    full = lambda *shape: pl.BlockSpec(shape, lambda: tuple(0 for _ in shape))
    return pl.pallas_call(
        _headb_body,
        in_specs=[
            full(B, N, 128),
            full(256, 512), full(512, 256), full(1, 256),
            full(256, 128), full(1, 128),
            full(128, 32), full(1, 32),
            full(32, 40), full(1, 40),
        ],
        out_specs=full(B, 40),
        out_shape=jax.ShapeDtypeStruct((B, 40), jnp.float32),
    )


def _edge_layer(xf, wat, wbt, c, o):
    u, v = _uv_call(c, o)(xf, wat, wbt)
    gidx = _knn_call(c)(xf, xf)
    ymax, part = _sc_gather(u, v, gidx, o)
    return _fin_call(o)(ymax, part)


def kernel(x, W1, W2, W3, W4, W5, L1, L2w, L2b, L3w, L3b, L4w, L4b, L5w, L5b):
    f32 = jnp.float32
    x0 = jnp.transpose(x, (0, 2, 1)).reshape(BN, 3)
    x0 = jnp.pad(x0, ((0, 0), (0, 5)))
    wat1 = jnp.pad(jnp.transpose(W1[:, :3]), ((0, 5), (0, 0)))
    wbt1 = jnp.pad(jnp.transpose(W1[:, 3:]), ((0, 5), (0, 0)))

    x1 = _edge_layer(x0, wat1, wbt1, 8, 64)
    x2 = _edge_layer(x1, jnp.transpose(W2[:, :64]), jnp.transpose(W2[:, 64:]),
                     64, 64)
    x3 = _edge_layer(x2, jnp.transpose(W3[:, :64]), jnp.transpose(W3[:, 64:]),
                     64, 128)
    x4 = _edge_layer(x3, jnp.transpose(W4[:, :128]), jnp.transpose(W4[:, 128:]),
                     128, 256)

    z = _heada_call()(
        x1, x2, x3, x4,
        jnp.transpose(W5[:, :64]), jnp.transpose(W5[:, 64:128]),
        jnp.transpose(W5[:, 128:256]), jnp.transpose(W5[:, 256:]),
    )
    out = _headb_call()(
        z.reshape(B, N, 128),
        jnp.transpose(L1), jnp.transpose(L2w), L2b.reshape(1, 256).astype(f32),
        jnp.transpose(L3w), L3b.reshape(1, 128),
        jnp.transpose(L4w), L4b.reshape(1, 32),
        jnp.transpose(L5w), L5b.reshape(1, 40),
    )
    return out
